# baseline (speedup 1.0000x reference)
import functools

import jax
import jax.numpy as jnp
from jax import lax
from jax.experimental import pallas as pl
from jax.experimental.pallas import tpu as pltpu
from jax.experimental.pallas import tpu_sc as plsc

B = 4096
D = 256
K = 8192
TEMP = 0.05
BR = 512
COLT = 2048
NBLK = B // BR
LOG2E = 1.4426950408889634
L = 16


def _sc_target_dots(targets, x, feats):
    info = plsc.get_sparse_core_info()
    nw = info.num_cores * info.num_subcores
    bpw = B // nw
    nch = D // L
    mesh = plsc.VectorSubcoreMesh(core_axis_name="c", subcore_axis_name="s")

    @functools.partial(
        pl.kernel, mesh=mesh,
        out_type=(jax.ShapeDtypeStruct((B * L,), jnp.float32),
                  jax.ShapeDtypeStruct((B * L,), jnp.float32)),
        scratch_types=[
            pltpu.VMEM((bpw,), jnp.int32),
            pltpu.VMEM((bpw,), jnp.int32),
            pltpu.VMEM((bpw, D), jnp.float32),
            pltpu.VMEM((bpw, D), jnp.float32),
            pltpu.VMEM((bpw * L,), jnp.float32),
            pltpu.VMEM((bpw * L,), jnp.float32),
            pltpu.SemaphoreType.DMA,
        ],
    )
    def k(tgt_hbm, x_hbm, feats_hbm, pm_hbm, ph_hbm,
          idx_v, idx2_v, rows, xv, pm_v, ph_v, sem):
        wid = lax.axis_index("s") * info.num_cores + lax.axis_index("c")
        base = wid * bpw
        pltpu.sync_copy(tgt_hbm.at[pl.ds(base, bpw)], idx_v)
        for j in range(bpw // L):
            sl = pl.ds(j * L, L)
            idx2_v[sl] = idx_v[sl] + K
        gm = pltpu.async_copy(feats_hbm.at[idx_v], rows, sem)
        pltpu.sync_copy(x_hbm.at[pl.ds(base, bpw)], xv)
        gm.wait()

        def dot_rows(p_v):
            def row_pair(r2, carry):
                for rr in range(2):
                    r = r2 * 2 + rr
                    acc = xv[r, pl.ds(0, L)] * rows[r, pl.ds(0, L)]
                    for c in range(1, nch):
                        sl = pl.ds(c * L, L)
                        acc = acc + xv[r, sl] * rows[r, sl]
                    p_v[pl.ds(r * L, L)] = acc
                return carry
            lax.fori_loop(0, bpw // 2, row_pair, 0)

        dot_rows(pm_v)
        pltpu.async_copy(feats_hbm.at[idx2_v], rows, sem).wait()
        dot_rows(ph_v)
        pltpu.sync_copy(pm_v, pm_hbm.at[pl.ds(base * L, bpw * L)])
        pltpu.sync_copy(ph_v, ph_hbm.at[pl.ds(base * L, bpw * L)])

    return k(targets, x, feats)


def _lse_body(x_ref, feats_ref, lsesum_ref, inv_ref, fb_ref):
    i = pl.program_id(0)

    @pl.when(i == 0)
    def _cast():
        fb_ref[...] = feats_ref[...].astype(jnp.bfloat16)

    x = x_ref[...]
    norm = jnp.sqrt(jnp.sum(x * x, axis=1, keepdims=True))
    xn = x / jnp.maximum(norm, 1e-12)
    xnb = (xn * (LOG2E / TEMP)).astype(jnp.bfloat16)

    def tile_term(f_tile):
        l = lax.dot_general(xnb, f_tile, (((1,), (1,)), ((), ())),
                            preferred_element_type=jnp.float32)
        return jnp.sum(jnp.exp2(l), axis=1)

    acc_m = jnp.zeros((BR,), jnp.float32)
    acc_h = jnp.zeros((BR,), jnp.float32)
    for c in range(K // COLT):
        acc_m = acc_m + tile_term(fb_ref[pl.ds(c * COLT, COLT), :])
        acc_h = acc_h + tile_term(fb_ref[pl.ds(K + c * COLT, COLT), :])
    inv_ref[...] = 1.0 / (TEMP * jnp.maximum(norm[:, 0], 1e-12))
    block = jnp.sum(jnp.log(acc_m) + jnp.log(acc_h))

    @pl.when(i == 0)
    def _init():
        lsesum_ref[0, 0] = 0.0

    lsesum_ref[0, 0] += block


def _lse_call(x, feats, interpret=False):
    return pl.pallas_call(
        _lse_body,
        grid=(NBLK,),
        in_specs=[
            pl.BlockSpec((BR, D), lambda i: (i, 0)),
            pl.BlockSpec((2 * K, D), lambda i: (0, 0)),
        ],
        out_specs=[
            pl.BlockSpec((1, 1), lambda i: (0, 0), memory_space=pltpu.SMEM),
            pl.BlockSpec((BR,), lambda i: (i,)),
        ],
        out_shape=[
            jax.ShapeDtypeStruct((1, 1), jnp.float32),
            jax.ShapeDtypeStruct((B,), jnp.float32),
        ],
        scratch_shapes=[pltpu.VMEM((2 * K, D), jnp.bfloat16)],
        interpret=interpret,
    )(x, feats)


def _combine_body(pm_ref, ph_ref, inv16_ref, lsesum_ref, out_ref):
    t_total = jnp.sum((pm_ref[...] + ph_ref[...]) * inv16_ref[...])
    out_ref[0, 0] = (lsesum_ref[0, 0] - t_total) * (0.5 / B)


def _combine_call(pm, ph, inv16, lsesum, interpret=False):
    return pl.pallas_call(
        _combine_body,
        in_specs=[
            pl.BlockSpec(memory_space=pltpu.VMEM),
            pl.BlockSpec(memory_space=pltpu.VMEM),
            pl.BlockSpec(memory_space=pltpu.VMEM),
            pl.BlockSpec(memory_space=pltpu.SMEM),
        ],
        out_specs=pl.BlockSpec(memory_space=pltpu.SMEM),
        out_shape=jax.ShapeDtypeStruct((1, 1), jnp.float32),
        interpret=interpret,
    )(pm, ph, inv16, lsesum)


def kernel(inputs, targets, features):
    tgt = targets.astype(jnp.int32)
    lsesum, inv = _lse_call(inputs, features)
    pm, ph = _sc_target_dots(tgt, inputs, features)
    inv16 = jnp.repeat(inv, L)
    out = _combine_call(pm, ph, inv16, lsesum)
    return out[0, 0]


# speedup vs baseline: 1.0461x; 1.0461x over previous
import functools

import jax
import jax.numpy as jnp
from jax import lax
from jax.experimental import pallas as pl
from jax.experimental.pallas import tpu as pltpu
from jax.experimental.pallas import tpu_sc as plsc

B = 4096
D = 256
K = 8192
TEMP = 0.05
BR = 512
COLT = 2048
NBLK = B // BR
LOG2E = 1.4426950408889634
L = 16


def _sc_target_dots(targets, x, feats):
    info = plsc.get_sparse_core_info()
    nw = info.num_cores * info.num_subcores
    bpw = B // nw
    nch = D // L
    mesh = plsc.VectorSubcoreMesh(core_axis_name="c", subcore_axis_name="s")

    @functools.partial(
        pl.kernel, mesh=mesh,
        out_type=jax.ShapeDtypeStruct((2 * B * L,), jnp.float32),
        scratch_types=[
            pltpu.VMEM((bpw,), jnp.int32),
            pltpu.VMEM((bpw,), jnp.int32),
            pltpu.VMEM((bpw, D), jnp.float32),
            pltpu.VMEM((bpw, D), jnp.float32),
            pltpu.VMEM((bpw * L,), jnp.float32),
            pltpu.VMEM((bpw * L,), jnp.float32),
            pltpu.SemaphoreType.DMA,
        ],
    )
    def k(tgt_hbm, x_hbm, feats_hbm, p_hbm,
          idx_v, idx2_v, rows, xv, pm_v, ph_v, sem):
        wid = lax.axis_index("s") * info.num_cores + lax.axis_index("c")
        base = wid * bpw
        pltpu.sync_copy(tgt_hbm.at[pl.ds(base, bpw)], idx_v)
        for j in range(bpw // L):
            sl = pl.ds(j * L, L)
            idx2_v[sl] = idx_v[sl] + K
        gm = pltpu.async_copy(feats_hbm.at[idx_v], rows, sem)
        pltpu.sync_copy(x_hbm.at[pl.ds(base, bpw)], xv)
        gm.wait()

        def dot_rows(p_v):
            def row_pair(r2, carry):
                for rr in range(2):
                    r = r2 * 2 + rr
                    acc = xv[r, pl.ds(0, L)] * rows[r, pl.ds(0, L)]
                    for c in range(1, nch):
                        sl = pl.ds(c * L, L)
                        acc = acc + xv[r, sl] * rows[r, sl]
                    p_v[pl.ds(r * L, L)] = acc
                return carry
            lax.fori_loop(0, bpw // 2, row_pair, 0)

        dot_rows(pm_v)
        pltpu.async_copy(feats_hbm.at[idx2_v], rows, sem).wait()
        dot_rows(ph_v)
        pltpu.sync_copy(pm_v, p_hbm.at[pl.ds(base * L, bpw * L)])
        pltpu.sync_copy(ph_v, p_hbm.at[pl.ds(B * L + base * L, bpw * L)])

    return k(targets, x, feats)


def _lse_body(x_ref, feats_ref, lsesum_ref, inv_ref, fb_ref):
    i = pl.program_id(0)

    @pl.when(i == 0)
    def _cast():
        fb_ref[...] = feats_ref[...].astype(jnp.bfloat16)

    x = x_ref[...]
    norm = jnp.sqrt(jnp.sum(x * x, axis=1, keepdims=True))
    xn = x / jnp.maximum(norm, 1e-12)
    xnb = (xn * (LOG2E / TEMP)).astype(jnp.bfloat16)

    def tile_term(f_tile):
        l = lax.dot_general(xnb, f_tile, (((1,), (1,)), ((), ())),
                            preferred_element_type=jnp.float32)
        return jnp.sum(jnp.exp2(l), axis=1)

    acc_m = jnp.zeros((BR,), jnp.float32)
    acc_h = jnp.zeros((BR,), jnp.float32)
    for c in range(K // COLT):
        acc_m = acc_m + tile_term(fb_ref[pl.ds(c * COLT, COLT), :])
        acc_h = acc_h + tile_term(fb_ref[pl.ds(K + c * COLT, COLT), :])
    inv_ref[...] = 1.0 / (TEMP * jnp.maximum(norm[:, 0], 1e-12))
    block = jnp.sum(jnp.log(acc_m) + jnp.log(acc_h))

    @pl.when(i == 0)
    def _init():
        lsesum_ref[0, 0] = 0.0

    lsesum_ref[0, 0] += block


def _lse_call(x, feats, interpret=False):
    return pl.pallas_call(
        _lse_body,
        grid=(NBLK,),
        in_specs=[
            pl.BlockSpec((BR, D), lambda i: (i, 0)),
            pl.BlockSpec((2 * K, D), lambda i: (0, 0)),
        ],
        out_specs=[
            pl.BlockSpec((1, 1), lambda i: (0, 0), memory_space=pltpu.SMEM),
            pl.BlockSpec((BR,), lambda i: (i,)),
        ],
        out_shape=[
            jax.ShapeDtypeStruct((1, 1), jnp.float32),
            jax.ShapeDtypeStruct((B,), jnp.float32),
        ],
        scratch_shapes=[pltpu.VMEM((2 * K, D), jnp.bfloat16)],
        interpret=interpret,
    )(x, feats)


def _combine_body(p_ref, inv_ref, lsesum_ref, out_ref):
    pm = p_ref[pl.ds(0, B * L)].reshape(B * L // 128, 128)
    ph = p_ref[pl.ds(B * L, B * L)].reshape(B * L // 128, 128)
    gpr = 128 // L
    li = lax.broadcasted_iota(jnp.int32, (128, gpr), 0)
    gi = lax.broadcasted_iota(jnp.int32, (128, gpr), 1)
    w = (li // L == gi).astype(jnp.float32)
    s = lax.dot_general(pm + ph, w, (((1,), (0,)), ((), ())),
                        preferred_element_type=jnp.float32)
    t_total = jnp.sum(s * inv_ref[...])
    out_ref[0, 0] = (lsesum_ref[0, 0] - t_total) * (0.5 / B)


def _combine_call(p, inv, lsesum, interpret=False):
    return pl.pallas_call(
        _combine_body,
        in_specs=[
            pl.BlockSpec(memory_space=pltpu.VMEM),
            pl.BlockSpec(memory_space=pltpu.VMEM),
            pl.BlockSpec(memory_space=pltpu.SMEM),
        ],
        out_specs=pl.BlockSpec(memory_space=pltpu.SMEM),
        out_shape=jax.ShapeDtypeStruct((1, 1), jnp.float32),
        interpret=interpret,
    )(p, inv, lsesum)


def kernel(inputs, targets, features):
    tgt = targets.astype(jnp.int32)
    lsesum, inv = _lse_call(inputs, features)
    p = _sc_target_dots(tgt, inputs, features)
    out = _combine_call(p, inv.reshape(B * L // 128, 128 // L), lsesum)
    return out[0, 0]
